# Initial kernel scaffold; baseline (speedup 1.0000x reference)
import functools

import jax
import jax.numpy as jnp
from jax import lax
from jax.experimental import pallas as pl
from jax.experimental.pallas import tpu as pltpu
from jax.experimental.pallas import tpu_sc as plsc

NN = 100000
EE = 1600000
FF = 128
HH = 32
GG = 4000
EPSV = 1e-5

CH = 128
NCHUNK = EE // CH
ROWS_T = NN // 16

_mesh = plsc.VectorSubcoreMesh(core_axis_name="c", subcore_axis_name="s")


def _hist_body(dst_hbm, zer_hbm, out_hbm, hacc, didx, obuf):
    c = lax.axis_index("c")
    s = lax.axis_index("s")
    w = c * 16 + s

    for j in range(8):
        obuf[pl.ds(j * 16, 16)] = jnp.ones((16,), jnp.float32)
    pltpu.sync_copy(zer_hbm, hacc.at[pl.ds(s * ROWS_T, ROWS_T)])
    plsc.subcore_barrier()

    nch = jnp.where(w < NCHUNK % 32, NCHUNK // 32 + 1, NCHUNK // 32)

    def step(i, _):
        base = (w + i * 32) * CH
        pltpu.sync_copy(dst_hbm.at[pl.ds(base, CH)], didx)
        pltpu.sync_copy(obuf, hacc.at[didx], add=True)
        return 0

    lax.fori_loop(0, nch, step, 0)
    plsc.subcore_barrier()
    pltpu.sync_copy(hacc.at[pl.ds(s * ROWS_T, ROWS_T)],
                    out_hbm.at[c, pl.ds(s * ROWS_T, ROWS_T)])


_hist = pl.kernel(
    _hist_body,
    out_type=jax.ShapeDtypeStruct((2, NN), jnp.float32),
    mesh=_mesh,
    scratch_types=[
        pltpu.VMEM_SHARED((NN,), jnp.float32),
        pltpu.VMEM((CH,), jnp.int32),
        pltpu.VMEM((CH,), jnp.float32),
    ],
)


def _spmm_body(th2_hbm, src_hbm, dst_hbm, zer_hbm, out_hbm,
               acc, sbuf, gidx, didx, rows):
    c = lax.axis_index("c")
    s = lax.axis_index("s")

    for j in range(10):
        pltpu.sync_copy(zer_hbm, acc.at[pl.ds(s * ROWS_T + j * 625, 625)])
    plsc.subcore_barrier()

    nch = jnp.where(s < NCHUNK % 16, NCHUNK // 16 + 1, NCHUNK // 16)

    def step(i, _):
        base = (s + i * 16) * CH
        pltpu.sync_copy(src_hbm.at[pl.ds(base, CH)], sbuf)
        pltpu.sync_copy(dst_hbm.at[pl.ds(base, CH)], didx)
        for j in range(CH // 16):
            v = sbuf[pl.ds(j * 16, 16)]
            gidx[pl.ds(j * 16, 16)] = v * 2 + c
        pltpu.sync_copy(th2_hbm.at[gidx], rows)
        pltpu.sync_copy(rows, acc.at[didx], add=True)
        return 0

    lax.fori_loop(0, nch, step, 0)
    plsc.subcore_barrier()
    pltpu.sync_copy(acc.at[pl.ds(s * ROWS_T, ROWS_T)],
                    out_hbm.at[c, pl.ds(s * ROWS_T, ROWS_T)])


_spmm = pl.kernel(
    _spmm_body,
    out_type=jax.ShapeDtypeStruct((2, NN, 16), jnp.float32),
    mesh=_mesh,
    scratch_types=[
        pltpu.VMEM_SHARED((NN, 16), jnp.float32),
        pltpu.VMEM((CH,), jnp.int32),
        pltpu.VMEM((CH,), jnp.int32),
        pltpu.VMEM((CH,), jnp.int32),
        pltpu.VMEM((CH, 16), jnp.float32),
    ],
)


_PROWS = NN // 2
_PFULL = _PROWS // CH
_PREM = _PROWS - _PFULL * CH


def _pool_body(h4_hbm, batch_hbm, zer_hbm, out_hbm,
               pacc, bidx, rbuf, bidx2, rbuf2):
    c = lax.axis_index("c")
    s = lax.axis_index("s")

    pltpu.sync_copy(zer_hbm, pacc.at[pl.ds(s * (GG // 16), GG // 16)])
    plsc.subcore_barrier()

    nch = jnp.where(s < _PFULL % 16, _PFULL // 16 + 1, _PFULL // 16)

    def step(i, _):
        base = c * _PROWS + (s + i * 16) * CH
        pltpu.sync_copy(batch_hbm.at[pl.ds(base, CH)], bidx)
        pltpu.sync_copy(h4_hbm.at[pl.ds(base, CH)], rbuf)
        pltpu.sync_copy(rbuf, pacc.at[bidx], add=True)
        return 0

    lax.fori_loop(0, nch, step, 0)

    @pl.when(s == 15)
    def _():
        base = c * _PROWS + _PFULL * CH
        pltpu.sync_copy(batch_hbm.at[pl.ds(base, _PREM)], bidx2)
        pltpu.sync_copy(h4_hbm.at[pl.ds(base, _PREM)], rbuf2)
        pltpu.sync_copy(rbuf2, pacc.at[bidx2], add=True)

    plsc.subcore_barrier()
    pltpu.sync_copy(pacc.at[pl.ds(s * (GG // 16), GG // 16)],
                    out_hbm.at[c, pl.ds(s * (GG // 16), GG // 16)])


_pool = pl.kernel(
    _pool_body,
    out_type=jax.ShapeDtypeStruct((2, GG, HH), jnp.float32),
    mesh=_mesh,
    scratch_types=[
        pltpu.VMEM_SHARED((GG, HH), jnp.float32),
        pltpu.VMEM((CH,), jnp.int32),
        pltpu.VMEM((CH, HH), jnp.float32),
        pltpu.VMEM((_PREM,), jnp.int32),
        pltpu.VMEM((_PREM, HH), jnp.float32),
    ],
)


BLK = 2000
NB = NN // BLK


def _tc_input_body(x_ref, w_ref, deg_ref, out_ref):
    dinv = lax.rsqrt(deg_ref[...] + 1.0)
    hw = jnp.dot(x_ref[...], w_ref[...], preferred_element_type=jnp.float32)
    out_ref[...] = hw * dinv[:, None]


def _tc_input(x, W0, deg):
    return pl.pallas_call(
        _tc_input_body,
        grid=(NB,),
        in_specs=[
            pl.BlockSpec((BLK, FF), lambda i: (i, 0)),
            pl.BlockSpec((FF, HH), lambda i: (0, 0)),
            pl.BlockSpec((BLK,), lambda i: (i,)),
        ],
        out_specs=pl.BlockSpec((BLK, HH), lambda i: (i, 0)),
        out_shape=jax.ShapeDtypeStruct((NN, HH), jnp.float32),
    )(x, W0, deg)


def _bn_core(agg0_ref, agg1_ref, thp_ref, deg_ref):
    dinv = lax.rsqrt(deg_ref[...] + 1.0)
    a = jnp.concatenate([agg0_ref[0], agg1_ref[0]], axis=1)
    return (a + thp_ref[...]) * dinv[:, None], dinv


def _tc_bnmm_body(agg0_ref, agg1_ref, thp_ref, deg_ref, g_ref, be_ref, w_ref,
                  out_ref, acc_ref):
    p = pl.program_id(0)
    i = pl.program_id(1)

    v, dinv = _bn_core(agg0_ref, agg1_ref, thp_ref, deg_ref)

    @pl.when((p == 0) & (i == 0))
    def _():
        acc_ref[...] = jnp.zeros_like(acc_ref)

    @pl.when(p == 0)
    def _():
        acc_ref[0:1, :HH] += jnp.sum(v, axis=0, keepdims=True)
        acc_ref[1:2, :HH] += jnp.sum(v * v, axis=0, keepdims=True)

    @pl.when(p == 1)
    def _():
        mu = acc_ref[0:1, :HH] * (1.0 / NN)
        var = acc_ref[1:2, :HH] * (1.0 / NN) - mu * mu
        scale = g_ref[...][None, :] * lax.rsqrt(var + EPSV)
        h = jnp.maximum((v - mu) * scale + be_ref[...][None, :], 0.0)
        hw = jnp.dot(h, w_ref[...], preferred_element_type=jnp.float32)
        out_ref[...] = hw * dinv[:, None]


def _tc_bnmm(agg, thp, deg, g, be, Wn):
    return pl.pallas_call(
        _tc_bnmm_body,
        grid=(2, NB),
        in_specs=[
            pl.BlockSpec((1, BLK, 16), lambda p, i: (0, i, 0)),
            pl.BlockSpec((1, BLK, 16), lambda p, i: (1, i, 0)),
            pl.BlockSpec((BLK, HH), lambda p, i: (i, 0)),
            pl.BlockSpec((BLK,), lambda p, i: (i,)),
            pl.BlockSpec((HH,), lambda p, i: (0,)),
            pl.BlockSpec((HH,), lambda p, i: (0,)),
            pl.BlockSpec((HH, HH), lambda p, i: (0, 0)),
        ],
        out_specs=pl.BlockSpec((BLK, HH), lambda p, i: (i, 0)),
        out_shape=jax.ShapeDtypeStruct((NN, HH), jnp.float32),
        scratch_shapes=[pltpu.VMEM((8, 128), jnp.float32)],
    )(agg, agg, thp, deg, g, be, Wn)


def _tc_final_body(agg0_ref, agg1_ref, thp_ref, deg_ref, g_ref, be_ref,
                   out_ref, acc_ref):
    p = pl.program_id(0)
    i = pl.program_id(1)

    v, _ = _bn_core(agg0_ref, agg1_ref, thp_ref, deg_ref)

    @pl.when((p == 0) & (i == 0))
    def _():
        acc_ref[...] = jnp.zeros_like(acc_ref)

    @pl.when(p == 0)
    def _():
        acc_ref[0:1, :HH] += jnp.sum(v, axis=0, keepdims=True)
        acc_ref[1:2, :HH] += jnp.sum(v * v, axis=0, keepdims=True)

    @pl.when(p == 1)
    def _():
        mu = acc_ref[0:1, :HH] * (1.0 / NN)
        var = acc_ref[1:2, :HH] * (1.0 / NN) - mu * mu
        scale = g_ref[...][None, :] * lax.rsqrt(var + EPSV)
        out_ref[...] = jnp.maximum((v - mu) * scale + be_ref[...][None, :], 0.0)


def _tc_final(agg, thp, deg, g, be):
    return pl.pallas_call(
        _tc_final_body,
        grid=(2, NB),
        in_specs=[
            pl.BlockSpec((1, BLK, 16), lambda p, i: (0, i, 0)),
            pl.BlockSpec((1, BLK, 16), lambda p, i: (1, i, 0)),
            pl.BlockSpec((BLK, HH), lambda p, i: (i, 0)),
            pl.BlockSpec((BLK,), lambda p, i: (i,)),
            pl.BlockSpec((HH,), lambda p, i: (0,)),
            pl.BlockSpec((HH,), lambda p, i: (0,)),
        ],
        out_specs=pl.BlockSpec((BLK, HH), lambda p, i: (i, 0)),
        out_shape=jax.ShapeDtypeStruct((NN, HH), jnp.float32),
        scratch_shapes=[pltpu.VMEM((8, 128), jnp.float32)],
    )(agg, agg, thp, deg, g, be)


def _tc_head_body(p_ref, w1_ref, b1_ref, w2_ref, b2_ref, w3_ref, b3_ref,
                  out_ref):
    pool = p_ref[0] + p_ref[1]
    o = jnp.maximum(
        jnp.dot(pool, w1_ref[...], preferred_element_type=jnp.float32)
        + b1_ref[...][None, :], 0.0)
    o = jnp.maximum(
        jnp.dot(o, w2_ref[...], preferred_element_type=jnp.float32)
        + b2_ref[...][None, :], 0.0)
    out_ref[...] = (jnp.dot(o, w3_ref[...], preferred_element_type=jnp.float32)
                    + b3_ref[...][None, :])


def _tc_head(pooled, mW1, mb1, mW2, mb2, mW3, mb3):
    return pl.pallas_call(
        _tc_head_body,
        out_shape=jax.ShapeDtypeStruct((GG, 1), jnp.float32),
    )(pooled, mW1, mb1, mW2, mb2, mW3, mb3)


def kernel(x, edge_index, batch, W0, b0, W1, b1, W2, b2, W3, b3,
           g0, be0, g1, be1, g2, be2, g3, be3,
           mW1, mb1, mW2, mb2, mW3, mb3):
    src = edge_index[0]
    dst = edge_index[1]

    zer_n = jnp.zeros((ROWS_T,), jnp.float32)
    zer_r = jnp.zeros((625, 16), jnp.float32)
    zer_p = jnp.zeros((GG // 16, HH), jnp.float32)

    hist = _hist(dst, zer_n)
    deg = hist[0] + hist[1]

    th = _tc_input(x, W0, deg)
    for (g, be, Wn) in ((g0, be0, W1), (g1, be1, W2), (g2, be2, W3)):
        agg = _spmm(th.reshape(2 * NN, 16), src, dst, zer_r)
        th = _tc_bnmm(agg, th, deg, g, be, Wn)
    agg = _spmm(th.reshape(2 * NN, 16), src, dst, zer_r)
    h4 = _tc_final(agg, th, deg, g3, be3)

    pooled = _pool(h4, batch, zer_p)
    return _tc_head(pooled, mW1, mb1, mW2, mb2, mW3, mb3)


# speedup vs baseline: 22.8782x; 22.8782x over previous
import functools

import jax
import jax.numpy as jnp
from jax import lax
from jax.experimental import pallas as pl
from jax.experimental.pallas import tpu as pltpu
from jax.experimental.pallas import tpu_sc as plsc

NN = 100000
EE = 1600000
FF = 128
HH = 32
GG = 4000
EPSV = 1e-5

CH = 128
NCHUNK = EE // CH
ROWS_T = NN // 16

_mesh = plsc.VectorSubcoreMesh(core_axis_name="c", subcore_axis_name="s")
_sc_params = pltpu.CompilerParams(use_tc_tiling_on_sc=False)


NP = 100096
HROWS = NP // 16


HGRP = 16
_HFULL = NCHUNK // HGRP
_HTAIL = NCHUNK - _HFULL * HGRP


def _hist_body(dst2_hbm, zer_hbm, out_hbm, hacc, dbig, obuf, zbuf, sems):
    c = lax.axis_index("c")
    s = lax.axis_index("s")
    w = c * 16 + s

    for j in range(8):
        obuf[pl.ds(j * 16, 16)] = jnp.ones((16,), jnp.float32)
    pltpu.sync_copy(zer_hbm, zbuf)
    pltpu.sync_copy(zbuf, hacc.at[pl.ds(s * HROWS, HROWS)])
    plsc.subcore_barrier()

    ng = _HFULL // 32 + jnp.where(w < _HFULL % 32, 1, 0)
    g0 = (_HFULL // 32) * w + jnp.minimum(w, _HFULL % 32)

    def step(i, _):
        pltpu.sync_copy(dst2_hbm.at[pl.ds((g0 + i) * HGRP, HGRP)], dbig)
        for j in range(HGRP):
            pltpu.async_copy(obuf, hacc.at[dbig.at[j]], sems, add=True)
        for j in range(HGRP):
            pltpu.make_async_copy(obuf, hacc.at[dbig.at[j]], sems).wait()
        return 0

    lax.fori_loop(0, ng, step, 0)

    @pl.when(w == 31)
    def _():
        pltpu.sync_copy(dst2_hbm.at[pl.ds(_HFULL * HGRP, _HTAIL)],
                        dbig.at[pl.ds(0, _HTAIL)])
        for j in range(_HTAIL):
            pltpu.async_copy(obuf, hacc.at[dbig.at[j]], sems, add=True)
        for j in range(_HTAIL):
            pltpu.make_async_copy(obuf, hacc.at[dbig.at[j]], sems).wait()

    plsc.subcore_barrier()
    pltpu.sync_copy(hacc.at[pl.ds(s * HROWS, HROWS)], zbuf)
    pltpu.sync_copy(zbuf, out_hbm.at[pl.ds(c * NP + s * HROWS, HROWS)])


_hist = pl.kernel(
    _hist_body,
    out_type=jax.ShapeDtypeStruct((2 * NP,), jnp.float32),
    mesh=_mesh,
    compiler_params=_sc_params,
    scratch_types=[
        pltpu.VMEM_SHARED((NP,), jnp.float32),
        pltpu.VMEM((HGRP, CH), jnp.int32),
        pltpu.VMEM((CH,), jnp.float32),
        pltpu.VMEM((HROWS,), jnp.float32),
        pltpu.SemaphoreType.DMA,
    ],
)


GRP = 4
GED = GRP * CH
NGRP = EE // GED


def _spmm_body(th2_hbm, src2_hbm, dst2_hbm, zer_hbm, out_hbm,
               acc, sb, gb, db, db2, rows, zbuf,
               semi0, semi1, semg0, semg1, sems0, sems1):
    c = lax.axis_index("c")
    s = lax.axis_index("s")
    semi = (semi0, semi1)
    semg = (semg0, semg1)
    sems = (sems0, sems1)

    pltpu.sync_copy(zer_hbm, zbuf)
    for j in range(34):
        pltpu.sync_copy(zbuf, acc.at[pl.ds(s * HROWS + j * 184, 184)])
    plsc.subcore_barrier()

    ng = NGRP // 16 + jnp.where(s < NGRP % 16, 1, 0)
    g0 = (NGRP // 16) * s + jnp.minimum(s, NGRP % 16)

    def idx_load(b, g):
        rb = (g0 + g) * GRP
        pltpu.async_copy(src2_hbm.at[pl.ds(rb, GRP)], sb.at[b], semi[b])
        pltpu.async_copy(dst2_hbm.at[pl.ds(rb, GRP)], db.at[b], semi[b])

    def idx_wait(b, g):
        rb = (g0 + g) * GRP
        pltpu.make_async_copy(src2_hbm.at[pl.ds(rb, GRP)], sb.at[b],
                              semi[b]).wait()
        pltpu.make_async_copy(dst2_hbm.at[pl.ds(rb, GRP)], db.at[b],
                              semi[b]).wait()

    def drain_scatters(b):
        for j in range(GRP):
            pltpu.make_async_copy(rows.at[b, j], acc.at[db2.at[b, j]],
                                  sems[b]).wait()

    idx_load(0, 0)
    idx_load(1, 1)

    def pair(go, _):
        for b in range(2):
            g = go * 2 + b

            @pl.when(g < ng)
            def _():
                @pl.when(go >= 1)
                def _():
                    drain_scatters(b)
                idx_wait(b, g)
                for j in range(GRP):
                    for k in range(CH // 16):
                        v = sb[b, j, pl.ds(k * 16, 16)]
                        gb[b, j, pl.ds(k * 16, 16)] = v * 2 + c
                        db2[b, j, pl.ds(k * 16, 16)] = db[b, j,
                                                         pl.ds(k * 16, 16)]
                @pl.when(g + 2 < ng)
                def _():
                    idx_load(b, g + 2)
                for j in range(GRP):
                    pltpu.async_copy(th2_hbm.at[gb.at[b, j]], rows.at[b, j],
                                     semg[b])
                for j in range(GRP):
                    pltpu.make_async_copy(th2_hbm.at[gb.at[b, j]],
                                          rows.at[b, j], semg[b]).wait()
                for j in range(GRP):
                    pltpu.async_copy(rows.at[b, j], acc.at[db2.at[b, j]],
                                     sems[b], add=True)
        return 0

    lax.fori_loop(0, (ng + 1) // 2, pair, 0)
    drain_scatters(0)
    drain_scatters(1)

    plsc.subcore_barrier()
    for j in range(34):
        pltpu.sync_copy(acc.at[pl.ds(s * HROWS + j * 184, 184)], zbuf)
        pltpu.sync_copy(zbuf,
                        out_hbm.at[c, pl.ds(s * HROWS + j * 184, 184)])


_spmm = pl.kernel(
    _spmm_body,
    out_type=jax.ShapeDtypeStruct((2, NP, 16), jnp.float32),
    mesh=_mesh,
    compiler_params=_sc_params,
    scratch_types=[
        pltpu.VMEM_SHARED((NP, 16), jnp.float32),
        pltpu.VMEM((2, GRP, CH), jnp.int32),
        pltpu.VMEM((2, GRP, CH), jnp.int32),
        pltpu.VMEM((2, GRP, CH), jnp.int32),
        pltpu.VMEM((2, GRP, CH), jnp.int32),
        pltpu.VMEM((2, GRP, CH, 16), jnp.float32),
        pltpu.VMEM((184, 16), jnp.float32),
        pltpu.SemaphoreType.DMA,
        pltpu.SemaphoreType.DMA,
        pltpu.SemaphoreType.DMA,
        pltpu.SemaphoreType.DMA,
        pltpu.SemaphoreType.DMA,
        pltpu.SemaphoreType.DMA,
    ],
)


GP = 4096

BLK = 2000
NB = NN // BLK


def _tc_input_body(x_ref, w_ref, deg_ref, out_ref):
    dinv = lax.rsqrt(deg_ref[0, 0, :] + 1.0)
    hw = jnp.dot(x_ref[...], w_ref[...], preferred_element_type=jnp.float32)
    out_ref[...] = hw * dinv[:, None]


def _tc_input(x, W0, deg):
    return pl.pallas_call(
        _tc_input_body,
        grid=(NB,),
        in_specs=[
            pl.BlockSpec((BLK, FF), lambda i: (i, 0)),
            pl.BlockSpec((FF, HH), lambda i: (0, 0)),
            pl.BlockSpec((1, 1, BLK), lambda i: (i, 0, 0)),
        ],
        out_specs=pl.BlockSpec((BLK, HH), lambda i: (i, 0)),
        out_shape=jax.ShapeDtypeStruct((NN, HH), jnp.float32),
    )(x, W0, deg)


def _bn_core(agg0_ref, agg1_ref, thp_ref, deg_ref):
    dinv = lax.rsqrt(deg_ref[0, 0, :] + 1.0)
    a = jnp.concatenate([agg0_ref[0], agg1_ref[0]], axis=1)
    return (a + thp_ref[...]) * dinv[:, None], dinv


def _tc_bnmm_body(agg0_ref, agg1_ref, thp_ref, deg_ref, g_ref, be_ref, w_ref,
                  out_ref, acc_ref):
    p = pl.program_id(0)
    i = pl.program_id(1)

    v, dinv = _bn_core(agg0_ref, agg1_ref, thp_ref, deg_ref)

    @pl.when((p == 0) & (i == 0))
    def _():
        acc_ref[...] = jnp.zeros_like(acc_ref)

    @pl.when(p == 0)
    def _():
        acc_ref[0:1, :HH] += jnp.sum(v, axis=0, keepdims=True)
        acc_ref[1:2, :HH] += jnp.sum(v * v, axis=0, keepdims=True)

    @pl.when(p == 1)
    def _():
        mu = acc_ref[0:1, :HH] * (1.0 / NN)
        var = acc_ref[1:2, :HH] * (1.0 / NN) - mu * mu
        scale = g_ref[...][None, :] * lax.rsqrt(var + EPSV)
        h = jnp.maximum((v - mu) * scale + be_ref[...][None, :], 0.0)
        hw = jnp.dot(h, w_ref[...], preferred_element_type=jnp.float32)
        out_ref[...] = hw * dinv[:, None]


def _tc_bnmm(agg, thp, deg, g, be, Wn):
    return pl.pallas_call(
        _tc_bnmm_body,
        grid=(2, NB),
        in_specs=[
            pl.BlockSpec((1, BLK, 16), lambda p, i: (0, i, 0)),
            pl.BlockSpec((1, BLK, 16), lambda p, i: (1, i, 0)),
            pl.BlockSpec((BLK, HH), lambda p, i: (i, 0)),
            pl.BlockSpec((1, 1, BLK), lambda p, i: (i, 0, 0)),
            pl.BlockSpec((HH,), lambda p, i: (0,)),
            pl.BlockSpec((HH,), lambda p, i: (0,)),
            pl.BlockSpec((HH, HH), lambda p, i: (0, 0)),
        ],
        out_specs=pl.BlockSpec((BLK, HH), lambda p, i: (i, 0)),
        out_shape=jax.ShapeDtypeStruct((NN, HH), jnp.float32),
        scratch_shapes=[pltpu.VMEM((8, 128), jnp.float32)],
    )(agg, agg, thp, deg, g, be, Wn)


def _tc_final_body(agg0_ref, agg1_ref, thp_ref, deg_ref, batch_ref,
                   g_ref, be_ref, w1_ref, b1_ref, w2_ref, b2_ref,
                   w3_ref, b3_ref, out_ref, acc_ref, pacc_ref):
    p = pl.program_id(0)
    i = pl.program_id(1)

    v, _ = _bn_core(agg0_ref, agg1_ref, thp_ref, deg_ref)

    @pl.when((p == 0) & (i == 0))
    def _():
        acc_ref[...] = jnp.zeros_like(acc_ref)

    @pl.when(p == 0)
    def _():
        acc_ref[0:1, :HH] += jnp.sum(v, axis=0, keepdims=True)
        acc_ref[1:2, :HH] += jnp.sum(v * v, axis=0, keepdims=True)

    @pl.when(p == 1)
    def _():
        @pl.when(i == 0)
        def _():
            pacc_ref[...] = jnp.zeros_like(pacc_ref)

        mu = acc_ref[0:1, :HH] * (1.0 / NN)
        var = acc_ref[1:2, :HH] * (1.0 / NN) - mu * mu
        scale = g_ref[...][None, :] * lax.rsqrt(var + EPSV)
        h4 = jnp.maximum((v - mu) * scale + be_ref[...][None, :], 0.0)

        b_row = batch_ref[0, 0, :][None, :]
        for q in range(GP // 512):
            rows = jax.lax.broadcasted_iota(jnp.int32, (512, BLK), 0) + q * 512
            oh = jnp.where(rows == b_row, 1.0, 0.0)
            pacc_ref[pl.ds(q * 512, 512), :] += jnp.dot(
                oh, h4, preferred_element_type=jnp.float32)

        @pl.when(i == NB - 1)
        def _():
            pool = pacc_ref[pl.ds(0, GG), :]
            o = jnp.maximum(
                jnp.dot(pool, w1_ref[...], preferred_element_type=jnp.float32)
                + b1_ref[...][None, :], 0.0)
            o = jnp.maximum(
                jnp.dot(o, w2_ref[...], preferred_element_type=jnp.float32)
                + b2_ref[...][None, :], 0.0)
            out_ref[...] = (
                jnp.dot(o, w3_ref[...], preferred_element_type=jnp.float32)
                + b3_ref[...][None, :])


def _tc_final(agg, thp, deg, batch3, g, be, mW1, mb1, mW2, mb2, mW3, mb3):
    return pl.pallas_call(
        _tc_final_body,
        grid=(2, NB),
        in_specs=[
            pl.BlockSpec((1, BLK, 16), lambda p, i: (0, i, 0)),
            pl.BlockSpec((1, BLK, 16), lambda p, i: (1, i, 0)),
            pl.BlockSpec((BLK, HH), lambda p, i: (i, 0)),
            pl.BlockSpec((1, 1, BLK), lambda p, i: (i, 0, 0)),
            pl.BlockSpec((1, 1, BLK), lambda p, i: (i, 0, 0)),
            pl.BlockSpec((HH,), lambda p, i: (0,)),
            pl.BlockSpec((HH,), lambda p, i: (0,)),
            pl.BlockSpec((HH, HH), lambda p, i: (0, 0)),
            pl.BlockSpec((HH,), lambda p, i: (0,)),
            pl.BlockSpec((HH, HH), lambda p, i: (0, 0)),
            pl.BlockSpec((HH,), lambda p, i: (0,)),
            pl.BlockSpec((HH, 1), lambda p, i: (0, 0)),
            pl.BlockSpec((1,), lambda p, i: (0,)),
        ],
        out_specs=pl.BlockSpec((GG, 1), lambda p, i: (0, 0)),
        out_shape=jax.ShapeDtypeStruct((GG, 1), jnp.float32),
        scratch_shapes=[pltpu.VMEM((8, 128), jnp.float32),
                        pltpu.VMEM((GP, HH), jnp.float32)],
    )(agg, agg, thp, deg, batch3, g, be, mW1, mb1, mW2, mb2, mW3, mb3)


def kernel(x, edge_index, batch, W0, b0, W1, b1, W2, b2, W3, b3,
           g0, be0, g1, be1, g2, be2, g3, be3,
           mW1, mb1, mW2, mb2, mW3, mb3):
    src = edge_index[0]
    dst = edge_index[1]

    zer_n = jnp.zeros((HROWS,), jnp.float32)
    zer_r = jnp.zeros((184, 16), jnp.float32)
    src2 = src.reshape(NCHUNK, CH)
    dst2 = dst.reshape(NCHUNK, CH)

    hist = _hist(dst2, zer_n)
    deg = (hist[:NN] + hist[NP:NP + NN]).reshape(NB, 1, BLK)

    th = _tc_input(x, W0, deg)
    for (g, be, Wn) in ((g0, be0, W1), (g1, be1, W2), (g2, be2, W3)):
        agg = _spmm(th.reshape(2 * NN, 16), src2, dst2, zer_r)
        th = _tc_bnmm(agg, th, deg, g, be, Wn)
    agg = _spmm(th.reshape(2 * NN, 16), src2, dst2, zer_r)
    batch3 = batch.reshape(NB, 1, BLK)
    return _tc_final(agg, th, deg, batch3, g3, be3,
                     mW1, mb1, mW2, mb2, mW3, mb3)
